# baseline (speedup 1.0000x reference)
import functools

import jax
import jax.numpy as jnp
from jax import lax
from jax.experimental import pallas as pl
from jax.experimental.pallas import tpu as pltpu
from jax.experimental.pallas import tpu_sc as plsc

_INFO = plsc.get_sparse_core_info()
_NC = _INFO.num_cores
_NS = _INFO.num_subcores
_NW = _NC * _NS

_U_PER_W, _U_BM = 3120, 624
_I_PER_W, _I_BM = 31248, 504


def _sc_copy_body(u_ref, i_ref, uo_ref, io_ref, buf, in_sems, out_sems):
    wid = lax.axis_index("s") * _NC + lax.axis_index("c")

    chunks = []
    for src, dst, per_w, bm in (
        (u_ref, uo_ref, _U_PER_W, _U_BM),
        (i_ref, io_ref, _I_PER_W, _I_BM),
    ):
        base = wid * per_w
        for k in range(per_w // bm):
            chunks.append((src, dst, base + k * bm, bm))
    n = len(chunks)

    def in_copy(c):
        src, _, off, bm = chunks[c]
        b = c % 2
        return pltpu.make_async_copy(
            src.at[pl.ds(pl.multiple_of(off, 8), bm), :],
            buf.at[b, pl.ds(0, bm)],
            in_sems.at[b],
        )

    def out_copy(c):
        _, dst, off, bm = chunks[c]
        b = c % 2
        return pltpu.make_async_copy(
            buf.at[b, pl.ds(0, bm)],
            dst.at[pl.ds(pl.multiple_of(off, 8), bm), :],
            out_sems.at[b],
        )

    in_copy(0).start()
    for c in range(n):
        in_copy(c).wait()
        out_copy(c).start()
        if c + 1 < n:
            if c >= 1:
                out_copy(c - 1).wait()
            in_copy(c + 1).start()
    out_copy(n - 1).wait()

    @pl.when(wid == _NW - 1)
    def _tails():
        for src, dst, off, rows in (
            (u_ref, uo_ref, _NW * _U_PER_W, 100000 - _NW * _U_PER_W),
            (i_ref, io_ref, _NW * _I_PER_W, 1000000 - _NW * _I_PER_W),
        ):
            pltpu.make_async_copy(
                src.at[pl.ds(off, rows), :],
                buf.at[0, pl.ds(0, rows)],
                in_sems.at[0],
            ).start()
            pltpu.make_async_copy(
                src.at[pl.ds(off, rows), :],
                buf.at[0, pl.ds(0, rows)],
                in_sems.at[0],
            ).wait()
            pltpu.make_async_copy(
                buf.at[0, pl.ds(0, rows)],
                dst.at[pl.ds(off, rows), :],
                out_sems.at[0],
            ).start()
            pltpu.make_async_copy(
                buf.at[0, pl.ds(0, rows)],
                dst.at[pl.ds(off, rows), :],
                out_sems.at[0],
            ).wait()


def kernel(user_emb, item_emb):
    mesh = plsc.VectorSubcoreMesh(core_axis_name="c", subcore_axis_name="s")
    f = functools.partial(
        pl.kernel,
        out_type=[
            jax.ShapeDtypeStruct(user_emb.shape, user_emb.dtype),
            jax.ShapeDtypeStruct(item_emb.shape, item_emb.dtype),
        ],
        mesh=mesh,
        compiler_params=pltpu.CompilerParams(use_tc_tiling_on_sc=False),
        scratch_types=[
            pltpu.VMEM((2, _U_BM, 64), jnp.float32),
            pltpu.SemaphoreType.DMA((2,)),
            pltpu.SemaphoreType.DMA((2,)),
        ],
    )(_sc_copy_body)
    u, i = f(user_emb, item_emb)
    return (u, i)


# speedup vs baseline: 1.2901x; 1.2901x over previous
import functools

import jax
import jax.numpy as jnp
from jax import lax
from jax.experimental import pallas as pl
from jax.experimental.pallas import tpu as pltpu
from jax.experimental.pallas import tpu_sc as plsc

_INFO = plsc.get_sparse_core_info()
_NC = _INFO.num_cores
_NS = _INFO.num_subcores
_NW = _NC * _NS

_U_PER_W, _U_BM = 3120, 240
_I_PER_W, _I_BM = 31248, 496


def _sc_copy_body(u_ref, i_ref, uo_ref, io_ref, buf, in_sems, out_sems):
    wid = lax.axis_index("s") * _NC + lax.axis_index("c")

    chunks = []
    for src, dst, per_w, bm in (
        (u_ref, uo_ref, _U_PER_W, _U_BM),
        (i_ref, io_ref, _I_PER_W, _I_BM),
    ):
        base = wid * per_w
        for k in range(per_w // bm):
            chunks.append((src, dst, base + k * bm, bm))
    n = len(chunks)

    def in_copy(c):
        src, _, off, bm = chunks[c]
        b = c % 2
        return pltpu.make_async_copy(
            src.at[pl.ds(pl.multiple_of(off, 8), bm), :],
            buf.at[b, pl.ds(0, bm)],
            in_sems.at[b],
        )

    def out_copy(c):
        _, dst, off, bm = chunks[c]
        b = c % 2
        return pltpu.make_async_copy(
            buf.at[b, pl.ds(0, bm)],
            dst.at[pl.ds(pl.multiple_of(off, 8), bm), :],
            out_sems.at[b],
        )

    in_copy(0).start()
    for c in range(n):
        in_copy(c).wait()
        out_copy(c).start()
        if c + 1 < n:
            if c >= 1:
                out_copy(c - 1).wait()
            in_copy(c + 1).start()
    out_copy(n - 1).wait()

    @pl.when(wid == _NW - 1)
    def _tails():
        for src, dst, off, rows in (
            (u_ref, uo_ref, _NW * _U_PER_W, 100000 - _NW * _U_PER_W),
            (i_ref, io_ref, _NW * _I_PER_W, 1000000 - _NW * _I_PER_W),
        ):
            pltpu.make_async_copy(
                src.at[pl.ds(off, rows), :],
                buf.at[0, pl.ds(0, rows)],
                in_sems.at[0],
            ).start()
            pltpu.make_async_copy(
                src.at[pl.ds(off, rows), :],
                buf.at[0, pl.ds(0, rows)],
                in_sems.at[0],
            ).wait()
            pltpu.make_async_copy(
                buf.at[0, pl.ds(0, rows)],
                dst.at[pl.ds(off, rows), :],
                out_sems.at[0],
            ).start()
            pltpu.make_async_copy(
                buf.at[0, pl.ds(0, rows)],
                dst.at[pl.ds(off, rows), :],
                out_sems.at[0],
            ).wait()


def kernel(user_emb, item_emb):
    mesh = plsc.VectorSubcoreMesh(core_axis_name="c", subcore_axis_name="s")
    f = functools.partial(
        pl.kernel,
        out_type=[
            jax.ShapeDtypeStruct(user_emb.shape, user_emb.dtype),
            jax.ShapeDtypeStruct(item_emb.shape, item_emb.dtype),
        ],
        mesh=mesh,
        scratch_types=[
            pltpu.VMEM((2, _I_BM, 64), jnp.float32),
            pltpu.SemaphoreType.DMA((2,)),
            pltpu.SemaphoreType.DMA((2,)),
        ],
    )(_sc_copy_body)
    u, i = f(user_emb, item_emb)
    return (u, i)


# speedup vs baseline: 1.3575x; 1.0522x over previous
import jax
import jax.numpy as jnp
from jax.experimental import pallas as pl
from jax.experimental.pallas import tpu as pltpu

_S = 4
_SB_I = 2500
_SB_U = 625
_DEPTH = 4
_NBUF = 8


def _copy_body(u_ref, i_ref, uo_ref, io_ref, buf, in_sems, out_sems):
    chunks = []
    for src, dst, sb in (
        (u_ref, uo_ref, _SB_U),
        (i_ref, io_ref, _SB_I),
    ):
        n, d = src.shape
        ws = src.reshape(_S, n // _S, d)
        wd = dst.reshape(_S, n // _S, d)
        for k in range((n // _S) // sb):
            chunks.append((ws, wd, k * sb, sb))
    n_chunks = len(chunks)

    def in_copy(c):
        src, _, off, sb = chunks[c]
        b = c % _NBUF
        return pltpu.make_async_copy(
            src.at[:, pl.ds(off, sb), :],
            buf.at[b, :, pl.ds(0, sb), :],
            in_sems.at[b],
        )

    def out_copy(c):
        _, dst, off, sb = chunks[c]
        b = c % _NBUF
        return pltpu.make_async_copy(
            buf.at[b, :, pl.ds(0, sb), :],
            dst.at[:, pl.ds(off, sb), :],
            out_sems.at[b],
        )

    for c in range(min(_DEPTH, n_chunks)):
        in_copy(c).start()
    for c in range(n_chunks):
        in_copy(c).wait()
        out_copy(c).start()
        nxt = c + _DEPTH
        if nxt < n_chunks:
            if nxt >= _NBUF:
                out_copy(nxt - _NBUF).wait()
            in_copy(nxt).start()
    for c in range(max(0, n_chunks - _NBUF), n_chunks):
        out_copy(c).wait()


def kernel(user_emb, item_emb):
    u, i = pl.pallas_call(
        _copy_body,
        in_specs=[
            pl.BlockSpec(memory_space=pl.ANY),
            pl.BlockSpec(memory_space=pl.ANY),
        ],
        out_specs=[
            pl.BlockSpec(memory_space=pl.ANY),
            pl.BlockSpec(memory_space=pl.ANY),
        ],
        out_shape=[
            jax.ShapeDtypeStruct(user_emb.shape, user_emb.dtype),
            jax.ShapeDtypeStruct(item_emb.shape, item_emb.dtype),
        ],
        scratch_shapes=[
            pltpu.VMEM((_NBUF, _S, _SB_I, 64), jnp.float32),
            pltpu.SemaphoreType.DMA((_NBUF,)),
            pltpu.SemaphoreType.DMA((_NBUF,)),
        ],
    )(user_emb, item_emb)
    return (u, i)
